# Initial kernel scaffold; baseline (speedup 1.0000x reference)
import jax
import jax.numpy as jnp
from jax.experimental import pallas as pl


def kernel(x, edge_index, batch, W_init, b_init, W_rel1, W_root1, b_rel1, W_rel2, W_root2, b_rel2, ggc_weight, W_ih, W_hh, b_ih, b_hh, W_lin, b_lin, W_lin1, b_lin1, W_lin2, b_lin2, W_lin3, b_lin3, W_lin4, b_lin4):
    raise NotImplementedError("write your pallas kernel here")


# speedup vs baseline: 4.3854x; 4.3854x over previous
import functools

import jax
import jax.numpy as jnp
from jax import lax
from jax.experimental import pallas as pl
from jax.experimental.pallas import tpu as pltpu
from jax.experimental.pallas import tpu_sc as plsc

F32 = jnp.float32
NCORE, NSUB = 2, 16


def _leaky(v):
    return jnp.where(v >= 0, v, 0.01 * v)


def _row_block(n):
    for r in (2000, 1024, 512, 256, 128, 64, 32, 16, 8):
        if n % r == 0:
            return r
    return n


def _tile8(b):
    return jnp.tile(b[None, :], (8, 1))


_DOT = dict(preferred_element_type=F32, precision=jax.lax.Precision.DEFAULT)


def _tc_init(xp, wfull):
    n, fin = xp.shape
    r = _row_block(n)

    def body(x_ref, w_ref, lo_ref, hi_ref):
        h = jnp.dot(x_ref[...], w_ref[...], **_DOT)
        lo_ref[...] = h[:, :32]
        hi_ref[...] = h[:, 32:]

    return pl.pallas_call(
        body,
        grid=(n // r,),
        in_specs=[pl.BlockSpec((r, fin), lambda i: (i, 0)),
                  pl.BlockSpec((fin, 64), lambda i: (0, 0))],
        out_specs=[pl.BlockSpec((r, 32), lambda i: (i, 0))] * 2,
        out_shape=[jax.ShapeDtypeStruct((n, 32), F32)] * 2,
    )(xp, wfull)


def _tc_graphconv(a_lo, a_hi, h_lo, h_hi, wrel_t, wroot_t, brel8):
    n = a_lo.shape[0]
    r = _row_block(n)

    def body(al, ah, hl, hh, wr, wo, b, ol, oh):
        agg = jnp.concatenate([al[...], ah[...]], axis=1)
        h = jnp.concatenate([hl[...], hh[...]], axis=1)
        out = jnp.dot(agg, wr[...], **_DOT) + jnp.dot(h, wo[...], **_DOT) + b[0:1, :]
        out = _leaky(out)
        ol[...] = out[:, :32]
        oh[...] = out[:, 32:]

    return pl.pallas_call(
        body,
        grid=(n // r,),
        in_specs=[pl.BlockSpec((r, 32), lambda i: (i, 0))] * 4 +
                 [pl.BlockSpec((64, 64), lambda i: (0, 0))] * 2 +
                 [pl.BlockSpec((8, 64), lambda i: (0, 0))],
        out_specs=[pl.BlockSpec((r, 32), lambda i: (i, 0))] * 2,
        out_shape=[jax.ShapeDtypeStruct((n, 32), F32)] * 2,
    )(a_lo, a_hi, h_lo, h_hi, wrel_t, wroot_t, brel8)


def _tc_gc2_fused(a_lo, a_hi, o_lo, o_hi, h0_lo, h0_hi, wrel_t, wroot_t, brel8, wg0):
    n = a_lo.shape[0]
    r = _row_block(n)

    def body(al, ah, ol, oh, zl, zh, wr, wo, b, wg, hl, hh, ml, mh):
        agg = jnp.concatenate([al[...], ah[...]], axis=1)
        o1 = jnp.concatenate([ol[...], oh[...]], axis=1)
        h0 = jnp.concatenate([zl[...], zh[...]], axis=1)
        out = jnp.dot(agg, wr[...], **_DOT) + jnp.dot(o1, wo[...], **_DOT) + b[0:1, :]
        h = _leaky(out + h0)
        m = jnp.dot(h, wg[...], **_DOT)
        hl[...] = h[:, :32]
        hh[...] = h[:, 32:]
        ml[...] = m[:, :32]
        mh[...] = m[:, 32:]

    return pl.pallas_call(
        body,
        grid=(n // r,),
        in_specs=[pl.BlockSpec((r, 32), lambda i: (i, 0))] * 6 +
                 [pl.BlockSpec((64, 64), lambda i: (0, 0))] * 2 +
                 [pl.BlockSpec((8, 64), lambda i: (0, 0))] +
                 [pl.BlockSpec((64, 64), lambda i: (0, 0))],
        out_specs=[pl.BlockSpec((r, 32), lambda i: (i, 0))] * 4,
        out_shape=[jax.ShapeDtypeStruct((n, 32), F32)] * 4,
    )(a_lo, a_hi, o_lo, o_hi, h0_lo, h0_hi, wrel_t, wroot_t, brel8, wg0)


def _gru(m, h, wih_t, whh_t, bih8, bhh8):
    gi = jnp.dot(m, wih_t, **_DOT) + bih8[0:1, :]
    gh = jnp.dot(h, whh_t, **_DOT) + bhh8[0:1, :]
    r = jax.nn.sigmoid(gi[:, :64] + gh[:, :64])
    z = jax.nn.sigmoid(gi[:, 64:128] + gh[:, 64:128])
    nn = jnp.tanh(gi[:, 128:] + r * gh[:, 128:])
    return (1.0 - z) * nn + z * h


def _tc_gru_mid(g_lo, g_hi, h_lo, h_hi, wih_t, whh_t, bih8, bhh8, wgn):
    n = g_lo.shape[0]
    r = _row_block(n)

    def body(gl, gh, hl, hh, wi, wh, bi, bh, wg, nl, nh, ml, mh):
        m = jnp.concatenate([gl[...], gh[...]], axis=1)
        h = jnp.concatenate([hl[...], hh[...]], axis=1)
        hn = _gru(m, h, wi[...], wh[...], bi[...], bh[...])
        mn = jnp.dot(hn, wg[...], **_DOT)
        nl[...] = hn[:, :32]
        nh[...] = hn[:, 32:]
        ml[...] = mn[:, :32]
        mh[...] = mn[:, 32:]

    return pl.pallas_call(
        body,
        grid=(n // r,),
        in_specs=[pl.BlockSpec((r, 32), lambda i: (i, 0))] * 4 +
                 [pl.BlockSpec((64, 192), lambda i: (0, 0))] * 2 +
                 [pl.BlockSpec((8, 192), lambda i: (0, 0))] * 2 +
                 [pl.BlockSpec((64, 64), lambda i: (0, 0))],
        out_specs=[pl.BlockSpec((r, 32), lambda i: (i, 0))] * 4,
        out_shape=[jax.ShapeDtypeStruct((n, 32), F32)] * 4,
    )(g_lo, g_hi, h_lo, h_hi, wih_t, whh_t, bih8, bhh8, wgn)


def _tc_gru_last(g_lo, g_hi, h_lo, h_hi, wih_t, whh_t, bih8, bhh8,
                 wlin_t, blin8, wlin1_tp, blin18):
    n = g_lo.shape[0]
    r = _row_block(n)

    def body(gl, gh, hl, hh, wi, wh, bi, bh, wl, bl, w1, b1, out):
        m = jnp.concatenate([gl[...], gh[...]], axis=1)
        h = jnp.concatenate([hl[...], hh[...]], axis=1)
        hn = _leaky(_gru(m, h, wi[...], wh[...], bi[...], bh[...]))
        f = _leaky(jnp.dot(hn, wl[...], **_DOT) + bl[0:1, :])
        f = _leaky(jnp.dot(f, w1[...], **_DOT) + b1[0:1, :])
        out[...] = f

    return pl.pallas_call(
        body,
        grid=(n // r,),
        in_specs=[pl.BlockSpec((r, 32), lambda i: (i, 0))] * 4 +
                 [pl.BlockSpec((64, 192), lambda i: (0, 0))] * 2 +
                 [pl.BlockSpec((8, 192), lambda i: (0, 0))] * 2 +
                 [pl.BlockSpec((64, 32), lambda i: (0, 0)),
                  pl.BlockSpec((8, 32), lambda i: (0, 0)),
                  pl.BlockSpec((32, 32), lambda i: (0, 0)),
                  pl.BlockSpec((8, 32), lambda i: (0, 0))],
        out_specs=pl.BlockSpec((r, 32), lambda i: (i, 0)),
        out_shape=jax.ShapeDtypeStruct((n, 32), F32),
    )(g_lo, g_hi, h_lo, h_hi, wih_t, whh_t, bih8, bhh8,
      wlin_t, blin8, wlin1_tp, blin18)


def _tc_head(amax2, asum2, w2t, b28, w3t, b38, w4tp, b48, nb, segp):
    def body2(am, ac, w2, b2, w3, b3, w4, b4, out):
        a = jnp.maximum(am[0], am[1])[:nb, :24]
        a = jnp.where(a == -jnp.inf, 0.0, a)
        c = (ac[0] + ac[1])[:nb, :24]
        g = jnp.concatenate([a, c], axis=1)
        g = _leaky(jnp.dot(g, w2[...], **_DOT) + b2[0:1, :])
        g = _leaky(jnp.dot(g, w3[...], **_DOT) + b3[0:1, :])
        g = _leaky(jnp.dot(g, w4[...], **_DOT) + b4[0:1, :])
        out[...] = g

    return pl.pallas_call(
        body2,
        out_shape=jax.ShapeDtypeStruct((nb, 8), F32),
    )(amax2, asum2, w2t, b28, w3t, b38, w4tp, b48)


_IB = 16


def _sc_segsum(x_lo, x_hi, src_t, dst_t):
    n = x_lo.shape[0]
    nchunk = src_t.shape[1]
    nblk = nchunk // _IB
    zpt = -(-(n + 1) // (NSUB * 64)) * 64
    acc_rows = NSUB * zpt
    zb = zpt // 32
    wba = -(-n // (NSUB * 8)) * 8
    wbl = n - (NSUB - 1) * wba
    mesh = plsc.VectorSubcoreMesh(core_axis_name="c", subcore_axis_name="s",
                                  num_cores=NCORE, num_subcores=NSUB)

    @functools.partial(
        pl.kernel,
        out_type=[jax.ShapeDtypeStruct((n, 32), F32)] * 2,
        mesh=mesh,
        scratch_types=[
            pltpu.VMEM((_IB, 128), jnp.int32),
            pltpu.VMEM((_IB, 128), jnp.int32),
            pltpu.VMEM((128, 32), F32),
            pltpu.VMEM((128, 32), F32),
            pltpu.VMEM((zb, 32), F32),
            pltpu.VMEM_SHARED((acc_rows, 32), F32),
            pltpu.SemaphoreType.DMA,
            pltpu.SemaphoreType.DMA,
        ],
        compiler_params=pltpu.CompilerParams(use_tc_tiling_on_sc=False),
    )
    def k(xlo_hbm, xhi_hbm, src_hbm, dst_hbm, olo_hbm, ohi_hbm,
          src_v, dst_v, rows0, rows1, zero_v, acc_sh, sem0, sem1):
        c = lax.axis_index("c")
        s = lax.axis_index("s")
        zv = jnp.zeros((16,), F32)

        def zb_body(i, carry):
            zero_v[i, 0:16] = zv
            zero_v[i, 16:32] = zv
            return carry

        lax.fori_loop(0, zb, zb_body, 0)
        for kk in range(32):
            pltpu.sync_copy(zero_v, acc_sh.at[pl.ds(s * zpt + kk * zb, zb)])
        plsc.subcore_barrier()

        def run(x_hbm):
            def body(ob, carry):
                pltpu.sync_copy(src_hbm.at[s, pl.ds(ob * _IB, _IB)], src_v)
                pltpu.sync_copy(dst_hbm.at[s, pl.ds(ob * _IB, _IB)], dst_v)
                bufs = (rows0, rows1)
                sems = (sem0, sem1)
                pltpu.async_copy(x_hbm.at[src_v.at[0]], bufs[0], sems[0]).wait()
                for j in range(_IB):
                    cur = bufs[j % 2]
                    if j + 1 < _IB:
                        nxt_cp = pltpu.async_copy(
                            x_hbm.at[src_v.at[j + 1]],
                            bufs[(j + 1) % 2], sems[(j + 1) % 2])
                    pltpu.sync_copy(cur, acc_sh.at[dst_v.at[j]], add=True)
                    if j + 1 < _IB:
                        nxt_cp.wait()
                return carry
            lax.fori_loop(0, nblk, body, 0)

        @pl.when(c == 0)
        def _():
            run(xlo_hbm)

        @pl.when(c == 1)
        def _():
            run(xhi_hbm)

        plsc.subcore_barrier()

        def wback(o_hbm):
            @pl.when(s < NSUB - 1)
            def _():
                pltpu.sync_copy(acc_sh.at[pl.ds(s * wba, wba)],
                                o_hbm.at[pl.ds(s * wba, wba)])

            if wbl > 0:
                @pl.when(s == NSUB - 1)
                def _():
                    pltpu.sync_copy(acc_sh.at[pl.ds((NSUB - 1) * wba, wbl)],
                                    o_hbm.at[pl.ds((NSUB - 1) * wba, wbl)])

        @pl.when(c == 0)
        def _():
            wback(olo_hbm)

        @pl.when(c == 1)
        def _():
            wback(ohi_hbm)

    return k(x_lo, x_hi, src_t, dst_t)


def _sc_readout(hfeat_p, batch_g, segp):
    np_ = hfeat_p.shape[0]
    cpt = np_ // (NCORE * NSUB)
    gr = cpt // 16
    nhb = 7 if gr % 7 == 0 else 2
    grb = gr // nhb
    hrows = cpt // nhb
    assert batch_g.shape == (NCORE * NSUB, gr, 16)
    spt = segp // NSUB
    scb = spt // 8
    mesh = plsc.VectorSubcoreMesh(core_axis_name="c", subcore_axis_name="s",
                                  num_cores=NCORE, num_subcores=NSUB)

    @functools.partial(
        pl.kernel,
        out_type=[jax.ShapeDtypeStruct((NCORE, segp, 32), F32)] * 2,
        mesh=mesh,
        scratch_types=[
            pltpu.VMEM((hrows, 32), F32),
            pltpu.VMEM((gr, 16), jnp.int32),
            pltpu.VMEM((segp, 32), F32),
            pltpu.VMEM((segp, 32), F32),
            pltpu.VMEM((NSUB, 8, 32), F32),
            pltpu.VMEM((NSUB, 8, 32), F32),
            pltpu.VMEM((spt, 32), F32),
            pltpu.VMEM((spt, 32), F32),
            pltpu.VMEM_SHARED((NSUB, segp, 32), F32),
            pltpu.VMEM_SHARED((NSUB, segp, 32), F32),
        ],
        compiler_params=pltpu.CompilerParams(use_tc_tiling_on_sc=False),
    )
    def k(h_hbm, b_hbm, omax_hbm, osum_hbm,
          hv, bv, pmax, psum, cmax, csum, obm, obs, shmax, shsum):
        c = lax.axis_index("c")
        s = lax.axis_index("s")
        w = c * NSUB + s
        base = w * cpt
        pltpu.sync_copy(b_hbm.at[w], bv)
        ninf = jnp.full((16,), -jnp.inf, F32)
        zv = jnp.zeros((16,), F32)

        def init_body(i, carry):
            pmax[i, 0:16] = ninf
            pmax[i, 16:32] = ninf
            psum[i, 0:16] = zv
            psum[i, 16:32] = zv
            return carry

        lax.fori_loop(0, segp, init_body, 0)

        for blk in range(nhb):
            pltpu.sync_copy(h_hbm.at[pl.ds(base + blk * hrows, hrows)], hv)

            def grp_body(g, carry):
                segs = bv[g + blk * grb]
                for lane in range(16):
                    seg = segs[lane]
                    r = g * 16 + lane
                    h0 = hv[r, 0:16]
                    h1 = hv[r, 16:32]
                    pmax[seg, 0:16] = jnp.maximum(pmax[seg, 0:16], h0)
                    pmax[seg, 16:32] = jnp.maximum(pmax[seg, 16:32], h1)
                    psum[seg, 0:16] = psum[seg, 0:16] + h0
                    psum[seg, 16:32] = psum[seg, 16:32] + h1
                return carry

            lax.fori_loop(0, grb, grp_body, 0)

        pltpu.sync_copy(pmax, shmax.at[s])
        pltpu.sync_copy(psum, shsum.at[s])
        plsc.subcore_barrier()
        for cb in range(scb):
            for t in range(NSUB):
                pltpu.sync_copy(shmax.at[t, pl.ds(s * spt + cb * 8, 8)],
                                cmax.at[t])
                pltpu.sync_copy(shsum.at[t, pl.ds(s * spt + cb * 8, 8)],
                                csum.at[t])

            def comb_body(i, carry):
                m0 = cmax[0, i, 0:16]
                m1 = cmax[0, i, 16:32]
                s0 = csum[0, i, 0:16]
                s1 = csum[0, i, 16:32]
                for t in range(1, NSUB):
                    m0 = jnp.maximum(m0, cmax[t, i, 0:16])
                    m1 = jnp.maximum(m1, cmax[t, i, 16:32])
                    s0 = s0 + csum[t, i, 0:16]
                    s1 = s1 + csum[t, i, 16:32]
                obm[cb * 8 + i, 0:16] = m0
                obm[cb * 8 + i, 16:32] = m1
                obs[cb * 8 + i, 0:16] = s0
                obs[cb * 8 + i, 16:32] = s1
                return carry

            lax.fori_loop(0, 8, comb_body, 0)
        pltpu.sync_copy(obm, omax_hbm.at[c, pl.ds(s * spt, spt)])
        pltpu.sync_copy(obs, osum_hbm.at[c, pl.ds(s * spt, spt)])

    return k(hfeat_p, batch_g)


def kernel(x, edge_index, batch, W_init, b_init, W_rel1, W_root1, b_rel1,
           W_rel2, W_root2, b_rel2, ggc_weight, W_ih, W_hh, b_ih, b_hh,
           W_lin, b_lin, W_lin1, b_lin1, W_lin2, b_lin2, W_lin3, b_lin3,
           W_lin4, b_lin4):
    n, fin = x.shape
    e = edge_index.shape[1]
    nb = 512

    xp = jnp.concatenate(
        [x.astype(F32), jnp.ones((n, 1), F32), jnp.zeros((n, 16 - fin - 1), F32)],
        axis=1)
    wfull = jnp.zeros((16, 64), F32)
    wfull = wfull.at[:fin].set(W_init.T).at[fin].set(b_init)

    src = edge_index[0].astype(jnp.int32)
    dst = edge_index[1].astype(jnp.int32)
    nchunk = -(-(-(-e // (NSUB * 128))) // _IB) * _IB
    e_pad = NSUB * 128 * nchunk
    src_t = jnp.concatenate([src, jnp.zeros((e_pad - e,), jnp.int32)])
    dst_t = jnp.concatenate([dst, jnp.full((e_pad - e,), n, jnp.int32)])
    src_t = src_t.reshape(NSUB, nchunk, 128)
    dst_t = dst_t.reshape(NSUB, nchunk, 128)

    h0_lo, h0_hi = _tc_init(xp, wfull)
    a_lo, a_hi = _sc_segsum(h0_lo, h0_hi, src_t, dst_t)
    o_lo, o_hi = _tc_graphconv(a_lo, a_hi, h0_lo, h0_hi,
                               W_rel1.T, W_root1.T, _tile8(b_rel1))
    a_lo, a_hi = _sc_segsum(o_lo, o_hi, src_t, dst_t)
    h_lo, h_hi, m_lo, m_hi = _tc_gc2_fused(
        a_lo, a_hi, o_lo, o_hi, h0_lo, h0_hi,
        W_rel2.T, W_root2.T, _tile8(b_rel2), ggc_weight[0])

    wih_t, whh_t = W_ih.T, W_hh.T
    bih8, bhh8 = _tile8(b_ih), _tile8(b_hh)
    wlin1_tp = jnp.zeros((32, 32), F32).at[:, :24].set(W_lin1.T)
    blin18 = _tile8(jnp.zeros((32,), F32).at[:24].set(b_lin1))
    num_layers = ggc_weight.shape[0]
    hfeat = None
    for i in range(num_layers):
        g_lo, g_hi = _sc_segsum(m_lo, m_hi, src_t, dst_t)
        if i + 1 < num_layers:
            h_lo, h_hi, m_lo, m_hi = _tc_gru_mid(
                g_lo, g_hi, h_lo, h_hi, wih_t, whh_t, bih8, bhh8,
                ggc_weight[i + 1])
        else:
            hfeat = _tc_gru_last(
                g_lo, g_hi, h_lo, h_hi, wih_t, whh_t, bih8, bhh8,
                W_lin.T, _tile8(b_lin), wlin1_tp, blin18)

    np_ = -(-n // 1024) * 1024
    segp = -(-(nb + 1) // 128) * 128
    hfp = jnp.concatenate([hfeat, jnp.zeros((np_ - n, 32), F32)])
    bp = jnp.concatenate([batch.astype(jnp.int32),
                          jnp.full((np_ - n,), nb, jnp.int32)])
    bp = bp.reshape(NCORE * NSUB, -1, 16)
    amax2, asum2 = _sc_readout(hfp, bp, segp)

    w4tp = jnp.zeros((8, 8), F32).at[:, :1].set(W_lin4.T)
    b48 = _tile8(jnp.zeros((8,), F32).at[:1].set(b_lin4))
    g = _tc_head(amax2, asum2, W_lin2.T, _tile8(b_lin2),
                 W_lin3.T, _tile8(b_lin3), w4tp, b48, nb, segp)
    return g[:, :1]


# speedup vs baseline: 7.6593x; 1.7465x over previous
import functools

import jax
import jax.numpy as jnp
from jax import lax
from jax.experimental import pallas as pl
from jax.experimental.pallas import tpu as pltpu
from jax.experimental.pallas import tpu_sc as plsc

F32 = jnp.float32
NCORE, NSUB = 2, 16


def _leaky(v):
    return jnp.where(v >= 0, v, 0.01 * v)


def _row_block(n):
    for r in (5000, 2000, 1024, 512, 256, 128, 64, 32, 16, 8):
        if n % r == 0:
            return r
    return n


def _tile8(b):
    return jnp.tile(b[None, :], (8, 1))


_DOT = dict(preferred_element_type=F32, precision=jax.lax.Precision.DEFAULT)


def _tc_init(xp, wfull):
    n, fin = xp.shape
    r = _row_block(n)

    def body(x_ref, w_ref, lo_ref, hi_ref):
        h = jnp.dot(x_ref[...], w_ref[...], **_DOT)
        lo_ref[...] = h[:, :32]
        hi_ref[...] = h[:, 32:]

    return pl.pallas_call(
        body,
        grid=(n // r,),
        in_specs=[pl.BlockSpec((r, fin), lambda i: (i, 0)),
                  pl.BlockSpec((fin, 64), lambda i: (0, 0))],
        out_specs=[pl.BlockSpec((r, 32), lambda i: (i, 0))] * 2,
        out_shape=[jax.ShapeDtypeStruct((n, 32), F32)] * 2,
    )(xp, wfull)


def _tc_graphconv(a_lo, a_hi, h_lo, h_hi, wrel_t, wroot_t, brel8):
    n = a_lo.shape[0]
    r = _row_block(n)

    def body(al, ah, hl, hh, wr, wo, b, ol, oh):
        agg = jnp.concatenate([al[...], ah[...]], axis=1)
        h = jnp.concatenate([hl[...], hh[...]], axis=1)
        out = jnp.dot(agg, wr[...], **_DOT) + jnp.dot(h, wo[...], **_DOT) + b[0:1, :]
        out = _leaky(out)
        ol[...] = out[:, :32]
        oh[...] = out[:, 32:]

    return pl.pallas_call(
        body,
        grid=(n // r,),
        in_specs=[pl.BlockSpec((r, 32), lambda i: (i, 0))] * 4 +
                 [pl.BlockSpec((64, 64), lambda i: (0, 0))] * 2 +
                 [pl.BlockSpec((8, 64), lambda i: (0, 0))],
        out_specs=[pl.BlockSpec((r, 32), lambda i: (i, 0))] * 2,
        out_shape=[jax.ShapeDtypeStruct((n, 32), F32)] * 2,
    )(a_lo, a_hi, h_lo, h_hi, wrel_t, wroot_t, brel8)


def _tc_gc2_fused(a_lo, a_hi, o_lo, o_hi, h0_lo, h0_hi, wrel_t, wroot_t, brel8, wg0):
    n = a_lo.shape[0]
    r = _row_block(n)

    def body(al, ah, ol, oh, zl, zh, wr, wo, b, wg, hl, hh, ml, mh):
        agg = jnp.concatenate([al[...], ah[...]], axis=1)
        o1 = jnp.concatenate([ol[...], oh[...]], axis=1)
        h0 = jnp.concatenate([zl[...], zh[...]], axis=1)
        out = jnp.dot(agg, wr[...], **_DOT) + jnp.dot(o1, wo[...], **_DOT) + b[0:1, :]
        h = _leaky(out + h0)
        m = jnp.dot(h, wg[...], **_DOT)
        hl[...] = h[:, :32]
        hh[...] = h[:, 32:]
        ml[...] = m[:, :32]
        mh[...] = m[:, 32:]

    return pl.pallas_call(
        body,
        grid=(n // r,),
        in_specs=[pl.BlockSpec((r, 32), lambda i: (i, 0))] * 6 +
                 [pl.BlockSpec((64, 64), lambda i: (0, 0))] * 2 +
                 [pl.BlockSpec((8, 64), lambda i: (0, 0))] +
                 [pl.BlockSpec((64, 64), lambda i: (0, 0))],
        out_specs=[pl.BlockSpec((r, 32), lambda i: (i, 0))] * 4,
        out_shape=[jax.ShapeDtypeStruct((n, 32), F32)] * 4,
    )(a_lo, a_hi, o_lo, o_hi, h0_lo, h0_hi, wrel_t, wroot_t, brel8, wg0)


def _gru(m, h, wih_t, whh_t, bih8, bhh8):
    gi = jnp.dot(m, wih_t, **_DOT) + bih8[0:1, :]
    gh = jnp.dot(h, whh_t, **_DOT) + bhh8[0:1, :]
    r = jax.nn.sigmoid(gi[:, :64] + gh[:, :64])
    z = jax.nn.sigmoid(gi[:, 64:128] + gh[:, 64:128])
    nn = jnp.tanh(gi[:, 128:] + r * gh[:, 128:])
    return (1.0 - z) * nn + z * h


def _tc_gru_mid(g_lo, g_hi, h_lo, h_hi, wih_t, whh_t, bih8, bhh8, wgn):
    n = g_lo.shape[0]
    r = _row_block(n)

    def body(gl, gh, hl, hh, wi, wh, bi, bh, wg, nl, nh, ml, mh):
        m = jnp.concatenate([gl[...], gh[...]], axis=1)
        h = jnp.concatenate([hl[...], hh[...]], axis=1)
        hn = _gru(m, h, wi[...], wh[...], bi[...], bh[...])
        mn = jnp.dot(hn, wg[...], **_DOT)
        nl[...] = hn[:, :32]
        nh[...] = hn[:, 32:]
        ml[...] = mn[:, :32]
        mh[...] = mn[:, 32:]

    return pl.pallas_call(
        body,
        grid=(n // r,),
        in_specs=[pl.BlockSpec((r, 32), lambda i: (i, 0))] * 4 +
                 [pl.BlockSpec((64, 192), lambda i: (0, 0))] * 2 +
                 [pl.BlockSpec((8, 192), lambda i: (0, 0))] * 2 +
                 [pl.BlockSpec((64, 64), lambda i: (0, 0))],
        out_specs=[pl.BlockSpec((r, 32), lambda i: (i, 0))] * 4,
        out_shape=[jax.ShapeDtypeStruct((n, 32), F32)] * 4,
    )(g_lo, g_hi, h_lo, h_hi, wih_t, whh_t, bih8, bhh8, wgn)


def _tc_gru_last(g_lo, g_hi, h_lo, h_hi, wih_t, whh_t, bih8, bhh8,
                 wlin_t, blin8, wlin1_tp, blin18):
    n = g_lo.shape[0]
    r = _row_block(n)

    def body(gl, gh, hl, hh, wi, wh, bi, bh, wl, bl, w1, b1, out):
        m = jnp.concatenate([gl[...], gh[...]], axis=1)
        h = jnp.concatenate([hl[...], hh[...]], axis=1)
        hn = _leaky(_gru(m, h, wi[...], wh[...], bi[...], bh[...]))
        f = _leaky(jnp.dot(hn, wl[...], **_DOT) + bl[0:1, :])
        f = _leaky(jnp.dot(f, w1[...], **_DOT) + b1[0:1, :])
        out[...] = f

    return pl.pallas_call(
        body,
        grid=(n // r,),
        in_specs=[pl.BlockSpec((r, 32), lambda i: (i, 0))] * 4 +
                 [pl.BlockSpec((64, 192), lambda i: (0, 0))] * 2 +
                 [pl.BlockSpec((8, 192), lambda i: (0, 0))] * 2 +
                 [pl.BlockSpec((64, 32), lambda i: (0, 0)),
                  pl.BlockSpec((8, 32), lambda i: (0, 0)),
                  pl.BlockSpec((32, 32), lambda i: (0, 0)),
                  pl.BlockSpec((8, 32), lambda i: (0, 0))],
        out_specs=pl.BlockSpec((r, 32), lambda i: (i, 0)),
        out_shape=jax.ShapeDtypeStruct((n, 32), F32),
    )(g_lo, g_hi, h_lo, h_hi, wih_t, whh_t, bih8, bhh8,
      wlin_t, blin8, wlin1_tp, blin18)


def _tc_head(amax2, asum2, w2t, b28, w3t, b38, w4tp, b48, nb, segp):
    def body2(am, ac, w2, b2, w3, b3, w4, b4, out):
        a = jnp.maximum(am[0], am[1])[:nb, :24]
        a = jnp.where(a == -jnp.inf, 0.0, a)
        c = (ac[0] + ac[1])[:nb, :24]
        g = jnp.concatenate([a, c], axis=1)
        g = _leaky(jnp.dot(g, w2[...], **_DOT) + b2[0:1, :])
        g = _leaky(jnp.dot(g, w3[...], **_DOT) + b3[0:1, :])
        g = _leaky(jnp.dot(g, w4[...], **_DOT) + b4[0:1, :])
        out[...] = g

    return pl.pallas_call(
        body2,
        out_shape=jax.ShapeDtypeStruct((nb, 8), F32),
    )(amax2, asum2, w2t, b28, w3t, b38, w4tp, b48)


_IB = 8
_RING = 4


def _sc_segsum(x_lo, x_hi, src_t, dst_t):
    n = x_lo.shape[0]
    nchunk = src_t.shape[1]
    nblk = nchunk // _IB
    zpt = -(-(n + 1) // (NSUB * 64)) * 64
    acc_rows = NSUB * zpt
    zb = zpt // 32
    wba = -(-n // (NSUB * 8)) * 8
    wbl = n - (NSUB - 1) * wba
    mesh = plsc.VectorSubcoreMesh(core_axis_name="c", subcore_axis_name="s",
                                  num_cores=NCORE, num_subcores=NSUB)

    @functools.partial(
        pl.kernel,
        out_type=[jax.ShapeDtypeStruct((n, 32), F32)] * 2,
        mesh=mesh,
        scratch_types=[
            pltpu.VMEM((_IB, 128), jnp.int32),
            pltpu.VMEM((_IB, 128), jnp.int32),
            [pltpu.VMEM((128, 32), F32) for _ in range(_RING)],
            pltpu.VMEM((zb, 32), F32),
            pltpu.VMEM_SHARED((acc_rows, 32), F32),
            [pltpu.SemaphoreType.DMA for _ in range(_RING)],
            [pltpu.SemaphoreType.DMA for _ in range(_RING)],
        ],
        compiler_params=pltpu.CompilerParams(use_tc_tiling_on_sc=False),
    )
    def k(xlo_hbm, xhi_hbm, src_hbm, dst_hbm, olo_hbm, ohi_hbm,
          src_v, dst_v, bufs, zero_v, acc_sh, gsems, ssems):
        c = lax.axis_index("c")
        s = lax.axis_index("s")
        zv = jnp.zeros((16,), F32)

        def zb_body(i, carry):
            zero_v[i, 0:16] = zv
            zero_v[i, 16:32] = zv
            return carry

        lax.fori_loop(0, zb, zb_body, 0)
        for kk in range(32):
            pltpu.sync_copy(zero_v, acc_sh.at[pl.ds(s * zpt + kk * zb, zb)])
        plsc.subcore_barrier()

        def run(x_hbm):
            def body(ob, carry):
                pltpu.sync_copy(src_hbm.at[s, pl.ds(ob * _IB, _IB)], src_v)
                pltpu.sync_copy(dst_hbm.at[s, pl.ds(ob * _IB, _IB)], dst_v)
                gd = [pltpu.async_copy(x_hbm.at[src_v.at[j]], bufs[j], gsems[j])
                      for j in range(_RING)]
                sd = [None] * _RING
                for j in range(_IB):
                    b = j % _RING
                    gd[b].wait()
                    sd[b] = pltpu.async_copy(bufs[b], acc_sh.at[dst_v.at[j]],
                                             ssems[b], add=True)
                    if j + _RING < _IB:
                        sd[b].wait()
                        gd[b] = pltpu.async_copy(
                            x_hbm.at[src_v.at[j + _RING]], bufs[b], gsems[b])
                for b in range(min(_RING, _IB)):
                    sd[(_IB - 1 - b) % _RING].wait()
                return carry
            lax.fori_loop(0, nblk, body, 0)

        @pl.when(c == 0)
        def _():
            run(xlo_hbm)

        @pl.when(c == 1)
        def _():
            run(xhi_hbm)

        plsc.subcore_barrier()

        def wback(o_hbm):
            @pl.when(s < NSUB - 1)
            def _():
                pltpu.sync_copy(acc_sh.at[pl.ds(s * wba, wba)],
                                o_hbm.at[pl.ds(s * wba, wba)])

            if wbl > 0:
                @pl.when(s == NSUB - 1)
                def _():
                    pltpu.sync_copy(acc_sh.at[pl.ds((NSUB - 1) * wba, wbl)],
                                    o_hbm.at[pl.ds((NSUB - 1) * wba, wbl)])

        @pl.when(c == 0)
        def _():
            wback(olo_hbm)

        @pl.when(c == 1)
        def _():
            wback(ohi_hbm)

    return k(x_lo, x_hi, src_t, dst_t)


def _sc_readout(hfeat_p, batch_g, segp):
    np_ = hfeat_p.shape[0]
    cpt = np_ // (NCORE * NSUB)
    gr = cpt // 16
    nhb = 7 if gr % 7 == 0 else 2
    grb = gr // nhb
    hrows = cpt // nhb
    assert batch_g.shape == (NCORE * NSUB, gr, 16)
    spt = segp // NSUB
    scb = spt // 8
    mesh = plsc.VectorSubcoreMesh(core_axis_name="c", subcore_axis_name="s",
                                  num_cores=NCORE, num_subcores=NSUB)

    @functools.partial(
        pl.kernel,
        out_type=[jax.ShapeDtypeStruct((NCORE, segp, 32), F32)] * 2,
        mesh=mesh,
        scratch_types=[
            pltpu.VMEM((hrows, 32), F32),
            pltpu.VMEM((gr, 16), jnp.int32),
            pltpu.VMEM((segp, 32), F32),
            pltpu.VMEM((segp, 32), F32),
            pltpu.VMEM((NSUB, 8, 32), F32),
            pltpu.VMEM((NSUB, 8, 32), F32),
            pltpu.VMEM((spt, 32), F32),
            pltpu.VMEM((spt, 32), F32),
            pltpu.VMEM_SHARED((NSUB, segp, 32), F32),
            pltpu.VMEM_SHARED((NSUB, segp, 32), F32),
        ],
        compiler_params=pltpu.CompilerParams(use_tc_tiling_on_sc=False),
    )
    def k(h_hbm, b_hbm, omax_hbm, osum_hbm,
          hv, bv, pmax, psum, cmax, csum, obm, obs, shmax, shsum):
        c = lax.axis_index("c")
        s = lax.axis_index("s")
        w = c * NSUB + s
        base = w * cpt
        pltpu.sync_copy(b_hbm.at[w], bv)
        ninf = jnp.full((16,), -jnp.inf, F32)
        zv = jnp.zeros((16,), F32)

        def init_body(i, carry):
            pmax[i, 0:16] = ninf
            pmax[i, 16:32] = ninf
            psum[i, 0:16] = zv
            psum[i, 16:32] = zv
            return carry

        lax.fori_loop(0, segp, init_body, 0)

        for blk in range(nhb):
            pltpu.sync_copy(h_hbm.at[pl.ds(base + blk * hrows, hrows)], hv)

            def grp_body(g, carry):
                segs = bv[g + blk * grb]
                for lane in range(16):
                    seg = segs[lane]
                    r = g * 16 + lane
                    h0 = hv[r, 0:16]
                    h1 = hv[r, 16:32]
                    pmax[seg, 0:16] = jnp.maximum(pmax[seg, 0:16], h0)
                    pmax[seg, 16:32] = jnp.maximum(pmax[seg, 16:32], h1)
                    psum[seg, 0:16] = psum[seg, 0:16] + h0
                    psum[seg, 16:32] = psum[seg, 16:32] + h1
                return carry

            lax.fori_loop(0, grb, grp_body, 0)

        pltpu.sync_copy(pmax, shmax.at[s])
        pltpu.sync_copy(psum, shsum.at[s])
        plsc.subcore_barrier()
        for cb in range(scb):
            for t in range(NSUB):
                pltpu.sync_copy(shmax.at[t, pl.ds(s * spt + cb * 8, 8)],
                                cmax.at[t])
                pltpu.sync_copy(shsum.at[t, pl.ds(s * spt + cb * 8, 8)],
                                csum.at[t])

            def comb_body(i, carry):
                m0 = cmax[0, i, 0:16]
                m1 = cmax[0, i, 16:32]
                s0 = csum[0, i, 0:16]
                s1 = csum[0, i, 16:32]
                for t in range(1, NSUB):
                    m0 = jnp.maximum(m0, cmax[t, i, 0:16])
                    m1 = jnp.maximum(m1, cmax[t, i, 16:32])
                    s0 = s0 + csum[t, i, 0:16]
                    s1 = s1 + csum[t, i, 16:32]
                obm[cb * 8 + i, 0:16] = m0
                obm[cb * 8 + i, 16:32] = m1
                obs[cb * 8 + i, 0:16] = s0
                obs[cb * 8 + i, 16:32] = s1
                return carry

            lax.fori_loop(0, 8, comb_body, 0)
        pltpu.sync_copy(obm, omax_hbm.at[c, pl.ds(s * spt, spt)])
        pltpu.sync_copy(obs, osum_hbm.at[c, pl.ds(s * spt, spt)])

    return k(hfeat_p, batch_g)


def kernel(x, edge_index, batch, W_init, b_init, W_rel1, W_root1, b_rel1,
           W_rel2, W_root2, b_rel2, ggc_weight, W_ih, W_hh, b_ih, b_hh,
           W_lin, b_lin, W_lin1, b_lin1, W_lin2, b_lin2, W_lin3, b_lin3,
           W_lin4, b_lin4):
    n, fin = x.shape
    e = edge_index.shape[1]
    nb = 512

    xp = jnp.concatenate(
        [x.astype(F32), jnp.ones((n, 1), F32), jnp.zeros((n, 16 - fin - 1), F32)],
        axis=1)
    wfull = jnp.zeros((16, 64), F32)
    wfull = wfull.at[:fin].set(W_init.T).at[fin].set(b_init)

    src = edge_index[0].astype(jnp.int32)
    dst = edge_index[1].astype(jnp.int32)
    nchunk = -(-(-(-e // (NSUB * 128))) // _IB) * _IB
    e_pad = NSUB * 128 * nchunk
    src_t = jnp.concatenate([src, jnp.zeros((e_pad - e,), jnp.int32)])
    dst_t = jnp.concatenate([dst, jnp.full((e_pad - e,), n, jnp.int32)])
    src_t = src_t.reshape(NSUB, nchunk, 128)
    dst_t = dst_t.reshape(NSUB, nchunk, 128)

    h0_lo, h0_hi = _tc_init(xp, wfull)
    a_lo, a_hi = _sc_segsum(h0_lo, h0_hi, src_t, dst_t)
    o_lo, o_hi = _tc_graphconv(a_lo, a_hi, h0_lo, h0_hi,
                               W_rel1.T, W_root1.T, _tile8(b_rel1))
    a_lo, a_hi = _sc_segsum(o_lo, o_hi, src_t, dst_t)
    h_lo, h_hi, m_lo, m_hi = _tc_gc2_fused(
        a_lo, a_hi, o_lo, o_hi, h0_lo, h0_hi,
        W_rel2.T, W_root2.T, _tile8(b_rel2), ggc_weight[0])

    wih_t, whh_t = W_ih.T, W_hh.T
    bih8, bhh8 = _tile8(b_ih), _tile8(b_hh)
    wlin1_tp = jnp.zeros((32, 32), F32).at[:, :24].set(W_lin1.T)
    blin18 = _tile8(jnp.zeros((32,), F32).at[:24].set(b_lin1))
    num_layers = ggc_weight.shape[0]
    hfeat = None
    for i in range(num_layers):
        g_lo, g_hi = _sc_segsum(m_lo, m_hi, src_t, dst_t)
        if i + 1 < num_layers:
            h_lo, h_hi, m_lo, m_hi = _tc_gru_mid(
                g_lo, g_hi, h_lo, h_hi, wih_t, whh_t, bih8, bhh8,
                ggc_weight[i + 1])
        else:
            hfeat = _tc_gru_last(
                g_lo, g_hi, h_lo, h_hi, wih_t, whh_t, bih8, bhh8,
                W_lin.T, _tile8(b_lin), wlin1_tp, blin18)

    np_ = -(-n // 1024) * 1024
    segp = -(-(nb + 1) // 128) * 128
    hfp = jnp.concatenate([hfeat, jnp.zeros((np_ - n, 32), F32)])
    bp = jnp.concatenate([batch.astype(jnp.int32),
                          jnp.full((np_ - n,), nb, jnp.int32)])
    bp = bp.reshape(NCORE * NSUB, -1, 16)
    amax2, asum2 = _sc_readout(hfp, bp, segp)

    w4tp = jnp.zeros((8, 8), F32).at[:, :1].set(W_lin4.T)
    b48 = _tile8(jnp.zeros((8,), F32).at[:1].set(b_lin4))
    g = _tc_head(amax2, asum2, W_lin2.T, _tile8(b_lin2),
                 W_lin3.T, _tile8(b_lin3), w4tp, b48, nb, segp)
    return g[:, :1]
